# baseline (speedup 1.0000x reference)
import jax
import jax.numpy as jnp
from jax.experimental import pallas as pl
from jax.experimental.pallas import tpu as pltpu

B, C, H, W = 64, 768, 14, 14
NCLS = 1000
LOSS_OFFSET = C * NCLS * 0.2


def _gather_kernel(labels_ref, mask_ref, scales_ref, loss_ref):
    labels_v = labels_ref[...]
    mask = mask_ref[...]
    iota = jax.lax.broadcasted_iota(jnp.int32, (B, NCLS), 1)
    onehot = (iota == labels_v[:, None]).astype(jnp.float32)
    scales = jax.lax.dot_general(
        onehot, mask,
        dimension_numbers=(((1,), (1,)), ((), ())),
        preferred_element_type=jnp.float32,
    )
    scales_ref[...] = scales[:, None, :]
    l1 = jnp.sum(jnp.abs(mask))
    loss_ref[0, 0] = jnp.maximum(l1 - LOSS_OFFSET, 0.0)


def _mul_kernel(x_ref, scales_ref, out_ref):
    col = scales_ref[0, 0]
    out_ref[0] = x_ref[0] * col[:, None, None]


def kernel(x, labels, mask):
    scales3, loss = pl.pallas_call(
        _gather_kernel,
        out_shape=[
            jax.ShapeDtypeStruct((B, 1, C), jnp.float32),
            jax.ShapeDtypeStruct((1, 1), jnp.float32),
        ],
        out_specs=[
            pl.BlockSpec(memory_space=pltpu.VMEM),
            pl.BlockSpec(memory_space=pltpu.SMEM),
        ],
    )(labels, mask)

    out = pl.pallas_call(
        _mul_kernel,
        grid=(B,),
        in_specs=[
            pl.BlockSpec((1, C, H, W), lambda b: (b, 0, 0, 0)),
            pl.BlockSpec((1, 1, C), lambda b: (b, 0, 0)),
        ],
        out_specs=pl.BlockSpec((1, C, H, W), lambda b: (b, 0, 0, 0)),
        out_shape=jax.ShapeDtypeStruct((B, C, H, W), x.dtype),
    )(x, scales3)
    return out, loss[0, 0]


# speedup vs baseline: 2.1620x; 2.1620x over previous
import jax
import jax.numpy as jnp
from jax.experimental import pallas as pl
from jax.experimental.pallas import tpu as pltpu

B, C, H, W = 64, 768, 14, 14
HW = H * W
NCLS = 1000
LOSS_OFFSET = C * NCLS * 0.2

ROWS = B * C
RBLK = 512
NBLK = ROWS // RBLK


def _gather_kernel(labels_ref, mask_ref, scales_ref, loss_ref):
    labels_v = labels_ref[...]
    mask = mask_ref[...]
    iota = jax.lax.broadcasted_iota(jnp.int32, (B, NCLS), 1)
    onehot = (iota == labels_v[:, None]).astype(jnp.float32)
    scales = jax.lax.dot_general(
        onehot, mask,
        dimension_numbers=(((1,), (1,)), ((), ())),
        preferred_element_type=jnp.float32,
    )
    scales_ref[...] = scales
    l1 = jnp.sum(jnp.abs(mask))
    loss_ref[0, 0] = jnp.maximum(l1 - LOSS_OFFSET, 0.0)


def _mul_kernel(x_ref, s_ref, out_ref):
    out_ref[...] = x_ref[...] * s_ref[0]


def kernel(x, labels, mask):
    scales, loss = pl.pallas_call(
        _gather_kernel,
        out_shape=[
            jax.ShapeDtypeStruct((B, C), jnp.float32),
            jax.ShapeDtypeStruct((1, 1), jnp.float32),
        ],
        out_specs=[
            pl.BlockSpec(memory_space=pltpu.VMEM),
            pl.BlockSpec(memory_space=pltpu.SMEM),
        ],
    )(labels, mask)

    x2 = x.reshape(ROWS, HW)
    s3 = scales.reshape(NBLK, RBLK, 1)
    out = pl.pallas_call(
        _mul_kernel,
        grid=(NBLK,),
        in_specs=[
            pl.BlockSpec((RBLK, HW), lambda i: (i, 0)),
            pl.BlockSpec((1, RBLK, 1), lambda i: (i, 0, 0)),
        ],
        out_specs=pl.BlockSpec((RBLK, HW), lambda i: (i, 0)),
        out_shape=jax.ShapeDtypeStruct((ROWS, HW), x.dtype),
    )(x2, s3)
    return out.reshape(B, C, H, W), loss[0, 0]


# speedup vs baseline: 21.3385x; 9.8698x over previous
import jax
import jax.numpy as jnp
from jax.experimental import pallas as pl
from jax.experimental.pallas import tpu as pltpu

B, C, H, W = 64, 768, 14, 14
HW = H * W
NCLS = 1000
LOSS_OFFSET = C * NCLS * 0.2

HBLK = 14
NBLK = HW // HBLK


def _fused_kernel(labels_ref, mask_t_ref, x_ref, out_ref, loss_ref, scales_ref):
    @pl.when(pl.program_id(0) == 0)
    def _():
        labels_v = labels_ref[...]
        mask_t = mask_t_ref[...]
        iota = jax.lax.broadcasted_iota(jnp.int32, (B, NCLS), 1)
        onehot = (iota == labels_v[:, None]).astype(jnp.float32)
        scales_ref[...] = jax.lax.dot_general(
            onehot, mask_t,
            dimension_numbers=(((1,), (0,)), ((), ())),
            preferred_element_type=jnp.float32,
        )
        l1 = jnp.sum(jnp.abs(mask_t))
        loss_ref[0, 0] = jnp.maximum(l1 - LOSS_OFFSET, 0.0)

    out_ref[...] = x_ref[...] * scales_ref[...][None, :, :]


def kernel(x, labels, mask):
    xt = jnp.transpose(x, (2, 3, 0, 1)).reshape(HW, B, C)
    mask_t = mask.T
    out_t, loss = pl.pallas_call(
        _fused_kernel,
        grid=(NBLK,),
        in_specs=[
            pl.BlockSpec(memory_space=pltpu.VMEM),
            pl.BlockSpec((NCLS, C), lambda i: (0, 0)),
            pl.BlockSpec((HBLK, B, C), lambda i: (i, 0, 0)),
        ],
        out_specs=[
            pl.BlockSpec((HBLK, B, C), lambda i: (i, 0, 0)),
            pl.BlockSpec(memory_space=pltpu.SMEM),
        ],
        out_shape=[
            jax.ShapeDtypeStruct((HW, B, C), x.dtype),
            jax.ShapeDtypeStruct((1, 1), jnp.float32),
        ],
        scratch_shapes=[pltpu.VMEM((B, C), jnp.float32)],
    )(labels, mask_t, xt)
    out = jnp.transpose(out_t.reshape(H, W, B, C), (2, 3, 0, 1))
    return out, loss[0, 0]
